# baseline (speedup 1.0000x reference)
import functools
import math

import jax
import jax.numpy as jnp
from jax.experimental import pallas as pl

D = 256
H = 8
HID = 1024
L = 4
DH = D // H


def _body(x_ref, mem_ref, flags_ref, wq_ref, bq_ref, wk_ref, bk_ref,
          wv_ref, bv_ref, wo_ref, bo_ref, wf1_ref, bf1_ref, wf2_ref,
          bf2_ref, ws_ref, bs_ref, g1_ref, gb1_ref, g2_ref, gb2_ref,
          et_ref, e_ref, out_ref):
    f32 = jnp.float32
    bf16 = jnp.bfloat16
    x = x_ref[...]
    xb = x.astype(bf16)
    flags = flags_ref[...]
    scale = 1.0 / math.sqrt(DH)

    q = jnp.dot(xb, wq_ref[...], preferred_element_type=f32) + bq_ref[...]
    m = [mem_ref[:, l * D:(l + 1) * D] for l in range(L)]
    mb = [m[l].astype(bf16) for l in range(L)]
    k = [jnp.dot(mb[l], wk_ref[...], preferred_element_type=f32) + bk_ref[...]
         for l in range(L)]
    v = [jnp.dot(mb[l], wv_ref[...], preferred_element_type=f32) + bv_ref[...]
         for l in range(L)]

    et = et_ref[...]
    s = [jnp.dot((q * k[l]).astype(bf16), et, preferred_element_type=f32)
         * scale + flags[:, l:l + 1] for l in range(L)]
    mx = jnp.maximum(jnp.maximum(s[0], s[1]), jnp.maximum(s[2], s[3]))
    ex = [jnp.exp(s[l] - mx) for l in range(L)]
    den = ex[0] + ex[1] + ex[2] + ex[3]
    a = [ex[l] / den for l in range(L)]

    e_exp = e_ref[...]
    o = jnp.zeros_like(x)
    for l in range(L):
        o = o + jnp.dot(a[l].astype(bf16), e_exp,
                        preferred_element_type=f32) * v[l]
    o = jnp.dot(o.astype(bf16), wo_ref[...],
                preferred_element_type=f32) + bo_ref[...]

    def ln(y, g, b):
        mu = jnp.mean(y, axis=-1, keepdims=True)
        yc = y - mu
        var = jnp.mean(yc * yc, axis=-1, keepdims=True)
        return yc * jax.lax.rsqrt(var + 1e-5) * g + b

    e1 = ln(x + o, g1_ref[...], gb1_ref[...])
    h1 = jnp.maximum(
        jnp.dot(e1.astype(bf16), wf1_ref[...], preferred_element_type=f32)
        + bf1_ref[...], 0.0)
    e2 = jnp.dot(h1.astype(bf16), wf2_ref[...],
                 preferred_element_type=f32) + bf2_ref[...]
    e3 = ln(e1 + e2, g2_ref[...], gb2_ref[...])

    valid = flags[:, 4:5]
    saved = flags[:, 5:6]
    oe = jnp.where(valid > 0, e3, x)
    se = jnp.dot(oe.astype(bf16), ws_ref[...],
                 preferred_element_type=f32) + bs_ref[...]

    out_ref[:, 0:D] = oe
    for l in range(L - 1):
        out_ref[:, (l + 1) * D:(l + 2) * D] = jnp.where(
            saved > 0, m[l + 1], m[l])
    out_ref[:, L * D:(L + 1) * D] = jnp.where(saved > 0, se, m[L - 1])


@functools.partial(jax.jit, static_argnames=())
def kernel(output_embedding, scores, mem_padding_mask, save_period, mem_bank,
           save_proj_w, save_proj_b, in_proj_w, in_proj_b, out_proj_w,
           out_proj_b, fc1_w, fc1_b, fc2_w, fc2_b, ln1_g, ln1_b, ln2_g,
           ln2_b):
    f32 = jnp.float32
    n = output_embedding.shape[0]
    x = output_embedding
    mem2 = mem_bank.reshape(n, L * D)

    mask_add = jnp.where(mem_padding_mask, -1e9, 0.0).astype(f32)
    valid_f = (~mem_padding_mask[:, L - 1]).astype(f32)[:, None]
    saved_f = ((save_period == 0) & (scores > 0.0)).astype(f32)[:, None]
    flags = jnp.concatenate(
        [mask_add, valid_f, saved_f, jnp.zeros((n, 2), f32)], axis=1)

    wq = in_proj_w[:D].T
    wk = in_proj_w[D:2 * D].T
    wv = in_proj_w[2 * D:].T
    bq = in_proj_b[:D][None, :]
    bk = in_proj_b[D:2 * D][None, :]
    bv = in_proj_b[2 * D:][None, :]
    wo = out_proj_w.T
    bo = out_proj_b[None, :]
    wf1 = fc1_w.T
    bf1 = fc1_b[None, :]
    wf2 = fc2_w.T
    bf2 = fc2_b[None, :]
    ws = save_proj_w.T
    bs = save_proj_b[None, :]
    g1 = ln1_g[None, :]
    gb1 = ln1_b[None, :]
    g2 = ln2_g[None, :]
    gb2 = ln2_b[None, :]

    e_exp = jnp.repeat(jnp.eye(H, dtype=jnp.bfloat16), DH, axis=1)
    et = e_exp.T

    wq, wk, wv, wo, wf1, wf2, ws = (
        w.astype(jnp.bfloat16) for w in (wq, wk, wv, wo, wf1, wf2, ws))

    t = 512 if n % 512 == 0 else n
    grid = (n // t,)

    def row_spec(width):
        return pl.BlockSpec((t, width), lambda i: (i, 0))

    def const_spec(shape):
        return pl.BlockSpec(shape, lambda i: (0,) * len(shape))

    consts = [wq, bq, wk, bk, wv, bv, wo, bo, wf1, bf1, wf2, bf2, ws, bs,
              g1, gb1, g2, gb2, et, e_exp]
    out = pl.pallas_call(
        _body,
        grid=grid,
        in_specs=[row_spec(D), row_spec(L * D), row_spec(8)] +
                 [const_spec(c.shape) for c in consts],
        out_specs=row_spec((L + 1) * D),
        out_shape=jax.ShapeDtypeStruct((n, (L + 1) * D), f32),
    )(x, mem2, flags, *consts)
    return out.reshape(n, L + 1, D)


# speedup vs baseline: 1.0621x; 1.0621x over previous
import functools
import math

import jax
import jax.numpy as jnp
from jax.experimental import pallas as pl

D = 256
H = 8
HID = 1024
L = 4
DH = D // H


def _body(x_ref, mem_ref, flags_ref, wq_ref, bq_ref, wk_ref, bk_ref,
          wv_ref, bv_ref, wo_ref, bo_ref, wf1_ref, bf1_ref, wf2_ref,
          bf2_ref, ws_ref, bs_ref, g1_ref, gb1_ref, g2_ref, gb2_ref,
          et_ref, e_ref, out_ref):
    f32 = jnp.float32
    bf16 = jnp.bfloat16
    x = x_ref[...]
    xb = x.astype(bf16)
    flags = flags_ref[...]
    scale = 1.0 / math.sqrt(DH)

    q = jnp.dot(xb, wq_ref[...], preferred_element_type=f32) + bq_ref[...]
    m = [mem_ref[:, l, :] for l in range(L)]
    mb = [m[l].astype(bf16) for l in range(L)]
    k = [jnp.dot(mb[l], wk_ref[...], preferred_element_type=f32) + bk_ref[...]
         for l in range(L)]
    v = [jnp.dot(mb[l], wv_ref[...], preferred_element_type=f32) + bv_ref[...]
         for l in range(L)]

    et = et_ref[...]
    s = [jnp.dot((q * k[l]).astype(bf16), et, preferred_element_type=f32)
         * scale + flags[:, l:l + 1] for l in range(L)]
    mx = jnp.maximum(jnp.maximum(s[0], s[1]), jnp.maximum(s[2], s[3]))
    ex = [jnp.exp(s[l] - mx) for l in range(L)]
    den = ex[0] + ex[1] + ex[2] + ex[3]
    a = [ex[l] / den for l in range(L)]

    e_exp = e_ref[...]
    o = jnp.zeros_like(x)
    for l in range(L):
        o = o + jnp.dot(a[l].astype(bf16), e_exp,
                        preferred_element_type=f32) * v[l]
    o = jnp.dot(o.astype(bf16), wo_ref[...],
                preferred_element_type=f32) + bo_ref[...]

    def ln(y, g, b):
        mu = jnp.mean(y, axis=-1, keepdims=True)
        yc = y - mu
        var = jnp.mean(yc * yc, axis=-1, keepdims=True)
        return yc * jax.lax.rsqrt(var + 1e-5) * g + b

    e1 = ln(x + o, g1_ref[...], gb1_ref[...])
    h1 = jnp.maximum(
        jnp.dot(e1.astype(bf16), wf1_ref[...], preferred_element_type=f32)
        + bf1_ref[...], 0.0)
    e2 = jnp.dot(h1.astype(bf16), wf2_ref[...],
                 preferred_element_type=f32) + bf2_ref[...]
    e3 = ln(e1 + e2, g2_ref[...], gb2_ref[...])

    valid = flags[:, 4:5]
    saved = flags[:, 5:6]
    oe = jnp.where(valid > 0, e3, x)
    se = jnp.dot(oe.astype(bf16), ws_ref[...],
                 preferred_element_type=f32) + bs_ref[...]

    out_ref[:, 0, :] = oe
    for l in range(L - 1):
        out_ref[:, l + 1, :] = jnp.where(saved > 0, m[l + 1], m[l])
    out_ref[:, L, :] = jnp.where(saved > 0, se, m[L - 1])


@functools.partial(jax.jit, static_argnames=())
def kernel(output_embedding, scores, mem_padding_mask, save_period, mem_bank,
           save_proj_w, save_proj_b, in_proj_w, in_proj_b, out_proj_w,
           out_proj_b, fc1_w, fc1_b, fc2_w, fc2_b, ln1_g, ln1_b, ln2_g,
           ln2_b):
    f32 = jnp.float32
    n = output_embedding.shape[0]
    x = output_embedding

    mask_add = jnp.where(mem_padding_mask, -1e9, 0.0).astype(f32)
    valid_f = (~mem_padding_mask[:, L - 1]).astype(f32)[:, None]
    saved_f = ((save_period == 0) & (scores > 0.0)).astype(f32)[:, None]
    flags = jnp.concatenate(
        [mask_add, valid_f, saved_f, jnp.zeros((n, 2), f32)], axis=1)

    wq = in_proj_w[:D].T
    wk = in_proj_w[D:2 * D].T
    wv = in_proj_w[2 * D:].T
    bq = in_proj_b[:D][None, :]
    bk = in_proj_b[D:2 * D][None, :]
    bv = in_proj_b[2 * D:][None, :]
    wo = out_proj_w.T
    bo = out_proj_b[None, :]
    wf1 = fc1_w.T
    bf1 = fc1_b[None, :]
    wf2 = fc2_w.T
    bf2 = fc2_b[None, :]
    ws = save_proj_w.T
    bs = save_proj_b[None, :]
    g1 = ln1_g[None, :]
    gb1 = ln1_b[None, :]
    g2 = ln2_g[None, :]
    gb2 = ln2_b[None, :]

    e_exp = jnp.repeat(jnp.eye(H, dtype=jnp.bfloat16), DH, axis=1)
    et = e_exp.T

    wq, wk, wv, wo, wf1, wf2, ws = (
        w.astype(jnp.bfloat16) for w in (wq, wk, wv, wo, wf1, wf2, ws))

    t = 512 if n % 512 == 0 else n
    grid = (n // t,)

    def row_spec(width):
        return pl.BlockSpec((t, width), lambda i: (i, 0))

    def const_spec(shape):
        return pl.BlockSpec(shape, lambda i: (0,) * len(shape))

    consts = [wq, bq, wk, bk, wv, bv, wo, bo, wf1, bf1, wf2, bf2, ws, bs,
              g1, gb1, g2, gb2, et, e_exp]
    out = pl.pallas_call(
        _body,
        grid=grid,
        in_specs=[row_spec(D),
                  pl.BlockSpec((t, L, D), lambda i: (i, 0, 0)),
                  row_spec(8)] +
                 [const_spec(c.shape) for c in consts],
        out_specs=pl.BlockSpec((t, L + 1, D), lambda i: (i, 0, 0)),
        out_shape=jax.ShapeDtypeStruct((n, L + 1, D), f32),
    )(x, mem_bank, flags, *consts)
    return out


# speedup vs baseline: 1.4184x; 1.3355x over previous
import functools
import math

import jax
import jax.numpy as jnp
from jax.experimental import pallas as pl

D = 256
H = 8
HID = 1024
L = 4
DH = D // H


def _dgt(a, w):
    return jax.lax.dot_general(a, w, (((1,), (1,)), ((), ())),
                               preferred_element_type=jnp.float32)


def _body(x_ref, mem_ref, fl_ref, ipw_ref, ipb_ref, opw_ref, opb_ref,
          f1w_ref, f1b_ref, f2w_ref, f2b_ref, spw_ref, spb_ref,
          g1_ref, gb1_ref, g2_ref, gb2_ref, e_ref, out_ref):
    f32 = jnp.float32
    x = x_ref[...]
    scale = 1.0 / math.sqrt(DH)

    fl = jnp.transpose(fl_ref[0])

    wq = ipw_ref[0:D, :]
    wk = ipw_ref[D:2 * D, :]
    wv = ipw_ref[2 * D:3 * D, :]
    bq = ipb_ref[:, 0:D]
    bk = ipb_ref[:, D:2 * D]
    bv = ipb_ref[:, 2 * D:3 * D]

    q = _dgt(x, wq) + bq
    m = [mem_ref[:, l, :] for l in range(L)]
    k = [_dgt(m[l], wk) + bk for l in range(L)]
    v = [_dgt(m[l], wv) + bv for l in range(L)]

    e_exp = e_ref[...]
    s = [_dgt(q * k[l], e_exp) * scale + fl[:, l:l + 1] for l in range(L)]
    mx = jnp.maximum(jnp.maximum(s[0], s[1]), jnp.maximum(s[2], s[3]))
    ex = [jnp.exp(s[l] - mx) for l in range(L)]
    den = ex[0] + ex[1] + ex[2] + ex[3]
    a = [ex[l] / den for l in range(L)]

    o = jnp.zeros_like(x)
    for l in range(L):
        o = o + jnp.dot(a[l], e_exp, preferred_element_type=f32) * v[l]
    o = _dgt(o, opw_ref[...]) + opb_ref[...]

    def ln(y, g, b):
        mu = jnp.mean(y, axis=-1, keepdims=True)
        yc = y - mu
        var = jnp.mean(yc * yc, axis=-1, keepdims=True)
        return yc * jax.lax.rsqrt(var + 1e-5) * g + b

    e1 = ln(x + o, g1_ref[...], gb1_ref[...])
    h1 = jnp.maximum(_dgt(e1, f1w_ref[...]) + f1b_ref[...], 0.0)
    e2 = _dgt(h1, f2w_ref[...]) + f2b_ref[...]
    e3 = ln(e1 + e2, g2_ref[...], gb2_ref[...])

    valid = fl[:, 4:5]
    saved = fl[:, 5:6]
    oe = jnp.where(valid > 0, e3, x)
    se = _dgt(oe, spw_ref[...]) + spb_ref[...]

    out_ref[:, 0, :] = oe
    for l in range(L - 1):
        out_ref[:, l + 1, :] = jnp.where(saved > 0, m[l + 1], m[l])
    out_ref[:, L, :] = jnp.where(saved > 0, se, m[L - 1])


@functools.partial(jax.jit, static_argnames=())
def kernel(output_embedding, scores, mem_padding_mask, save_period, mem_bank,
           save_proj_w, save_proj_b, in_proj_w, in_proj_b, out_proj_w,
           out_proj_b, fc1_w, fc1_b, fc2_w, fc2_b, ln1_g, ln1_b, ln2_g,
           ln2_b):
    f32 = jnp.float32
    n = output_embedding.shape[0]
    x = output_embedding

    t = 512 if n % 512 == 0 else n
    g = n // t
    grid = (g,)

    ma = jnp.where(mem_padding_mask, -1e9, 0.0).astype(f32)
    valid_f = (~mem_padding_mask[:, L - 1]).astype(f32)
    saved_f = ((save_period == 0) & (scores > 0.0)).astype(f32)
    fl6 = jnp.stack([ma[:, 0].reshape(g, t), ma[:, 1].reshape(g, t),
                     ma[:, 2].reshape(g, t), ma[:, 3].reshape(g, t),
                     valid_f.reshape(g, t), saved_f.reshape(g, t)],
                    axis=1)

    ipb = in_proj_b[None, :]
    opb = out_proj_b[None, :]
    f1b = fc1_b[None, :]
    f2b = fc2_b[None, :]
    spb = save_proj_b[None, :]
    g1 = ln1_g[None, :]
    gb1 = ln1_b[None, :]
    g2 = ln2_g[None, :]
    gb2 = ln2_b[None, :]

    e_exp = jnp.repeat(jnp.eye(H, dtype=f32), DH, axis=1)

    def row_spec(width):
        return pl.BlockSpec((t, width), lambda i: (i, 0))

    def const_spec(shape):
        return pl.BlockSpec(shape, lambda i: (0,) * len(shape))

    consts = [in_proj_w, ipb, out_proj_w, opb, fc1_w, f1b, fc2_w, f2b,
              save_proj_w, spb, g1, gb1, g2, gb2, e_exp]
    out = pl.pallas_call(
        _body,
        grid=grid,
        in_specs=[row_spec(D),
                  pl.BlockSpec((t, L, D), lambda i: (i, 0, 0)),
                  pl.BlockSpec((1, 6, t), lambda i: (i, 0, 0))] +
                 [const_spec(c.shape) for c in consts],
        out_specs=pl.BlockSpec((t, L + 1, D), lambda i: (i, 0, 0)),
        out_shape=jax.ShapeDtypeStruct((n, L + 1, D), f32),
    )(x, mem_bank, fl6, *consts)
    return out
